# baseline (speedup 1.0000x reference)
import math

import jax
import jax.numpy as jnp
import numpy as np
from jax import lax
from jax.experimental import pallas as pl
from jax.experimental.pallas import tpu as pltpu
from jax.experimental.pallas import tpu_sc as plsc

BS, M, P = 32, 256, 20
PP = 24
N = BS * M
NB = 256
NBP = NB * PP

_SC_NC, _SC_NS = 2, 16
_SC_NW = _SC_NC * _SC_NS
_GB = (2 * N) // _SC_NW
_GCH = 128

_TY_C = np.arange(24) // 8
_RT_C = (np.arange(24) // 4) % 2
_TL_C = np.arange(24) % 4
_TLI_C = np.where(_TY_C == 1, np.array([2, 0, 0, 3])[_TL_C], _TL_C)


def _embed_gather_kernel(table_hbm, gidx_hbm, out_hbm, idx_v, rows_v, sem):
    wid = lax.axis_index("s") * _SC_NC + lax.axis_index("c")
    base = wid * _GB
    for c in range(_GB // _GCH):
        off = base + c * _GCH
        pltpu.sync_copy(gidx_hbm.at[pl.ds(off, _GCH)], idx_v)
        pltpu.async_copy(table_hbm.at[idx_v], rows_v, sem).wait()
        pltpu.sync_copy(rows_v, out_hbm.at[pl.ds(off, _GCH)])


def _encoder_kernel(pv_ref, vm_ref, sl_ref, hasl_ref, e1_ref, e2_ref,
                    w1a_ref, w2_ref, b2_ref,
                    w3a_ref, w3b_ref, b3_ref, w4_ref, b4_ref,
                    ffc_ref, ffs_ref, wf1c_ref, wf1s_ref, wf1l_ref, bf1_ref,
                    lng_ref, lnb_ref, wf2_ref, bf2_ref,
                    out1_ref, out2_ref):
    f32 = jnp.float32
    bf16 = jnp.bfloat16

    h1 = jnp.dot(pv_ref[...], w1a_ref[...], preferred_element_type=f32)
    h1 = jax.nn.relu(h1)

    h = jnp.dot(h1.astype(bf16), w2_ref[...], preferred_element_type=f32)
    h = h + b2_ref[...]
    vm = vm_ref[...]
    hm = h * vm

    io8 = jax.lax.broadcasted_iota(jnp.int32, (NB, 8, 1), 1)
    pbs = jnp.where(io8 >= P - 16, f32(-1e30), f32(0.0))

    hm3 = hm.reshape(NB, PP, 256)
    m8 = jnp.maximum(jnp.maximum(hm3[:, 0:8], hm3[:, 8:16]),
                     hm3[:, 16:24] + pbs)
    pooled = jnp.max(m8, axis=1)

    q = jnp.dot(pooled.astype(bf16), w3b_ref[...], preferred_element_type=f32)
    q = q + b3_ref[...]
    qb = jnp.broadcast_to(q.reshape(NB, 1, 256), (NB, PP, 256)).reshape(NBP, 256)
    t = jnp.dot(hm.astype(bf16), w3a_ref[...], preferred_element_type=f32) + qb
    t = jax.nn.relu(t)

    z = jnp.dot(t.astype(bf16), w4_ref[...], preferred_element_type=f32)
    z = (z + b4_ref[...]) * vm
    z3 = z.reshape(NB, PP, 128)
    z8 = jnp.maximum(jnp.maximum(z3[:, 0:8], z3[:, 8:16]),
                     z3[:, 16:24] + pbs)
    x_poly = jnp.max(z8, axis=1)

    sl = sl_ref[...]
    fh = (jnp.dot(ffc_ref[...], wf1c_ref[...], preferred_element_type=f32)
          + jnp.dot(ffs_ref[...], wf1s_ref[...], preferred_element_type=f32)
          + sl * wf1l_ref[...] + bf1_ref[...])
    mu = jnp.mean(fh, axis=-1, keepdims=True)
    var = jnp.mean((fh - mu) ** 2, axis=-1, keepdims=True)
    fh = (fh - mu) / jnp.sqrt(var + 1e-5) * lng_ref[...] + lnb_ref[...]
    fh = jax.nn.relu(fh)
    fh = jnp.dot(fh, wf2_ref[...], preferred_element_type=f32) + bf2_ref[...]
    xs = fh * hasl_ref[...]

    base = x_poly + xs
    out1_ref[...] = base + e1_ref[...]
    out2_ref[...] = base + e2_ref[...]


def kernel(polygon_center, polygon_speed_limit, point_position, point_vector,
           point_orientation, polygon_type, polygon_on_route, polygon_tl_status,
           polygon_has_speed_limit, valid_mask, W1, b1, g1, be1, m1, v1, W2, b2,
           W3, b3, g3, be3, m3, v3, W4, b4, freqs, Wf1, bf1, lng, lnb, Wf2, bf2,
           type_table, route_table, tl_table):
    f32 = jnp.float32
    bf16 = jnp.bfloat16

    o = point_orientation[:, :, 0]
    ones = jnp.ones(o.shape + (1,), f32)
    pv = jnp.concatenate(
        [point_position[:, :, 0] - polygon_center[:, :, None, :2],
         point_vector[:, :, 0],
         jnp.cos(o)[..., None], jnp.sin(o)[..., None], ones],
        axis=-1).reshape(N, P, 7)
    pv = jnp.pad(pv, ((0, 0), (0, PP - P), (0, 1))).reshape(N * PP, 8)
    vm = jnp.pad(valid_mask.astype(f32).reshape(N, P),
                 ((0, 0), (0, PP - P))).reshape(N * PP, 1)
    sl = polygon_speed_limit.reshape(N, 1)
    xf = sl * (freqs * (2.0 * math.pi))[None, :]
    ffc = jnp.cos(xf)
    ffs = jnp.sin(xf)
    hasl = polygon_has_speed_limit.reshape(N, 1).astype(f32)

    ty = polygon_type.reshape(N).astype(jnp.int32)
    rt = polygon_on_route.reshape(N).astype(jnp.int32)
    tl = polygon_tl_status.reshape(N).astype(jnp.int32)
    k = ty * 8 + rt * 4 + tl
    gidx = jnp.concatenate([k, k + 24])
    etab = jnp.concatenate([
        type_table[_TY_C] + route_table[_RT_C] + tl_table[_TL_C],
        type_table[_TY_C] + route_table[_RT_C] + tl_table[_TLI_C],
    ], axis=0)
    e12 = pl.kernel(
        _embed_gather_kernel,
        out_type=jax.ShapeDtypeStruct((2 * N, 128), f32),
        mesh=plsc.VectorSubcoreMesh(core_axis_name="c", subcore_axis_name="s"),
        scratch_types=[
            pltpu.VMEM((_GCH,), jnp.int32),
            pltpu.VMEM((_GCH, 128), f32),
            pltpu.SemaphoreType.DMA,
        ],
    )(etab, gidx)
    e1 = e12[0:N]
    e2 = e12[N:2 * N]

    s1 = g1 / jnp.sqrt(v1 + 1e-5)
    W1f = W1 * s1
    b1f = b1 * s1 + (be1 - m1 * s1)
    s3 = g3 / jnp.sqrt(v3 + 1e-5)
    W3f = W3 * s3
    b3f = b3 * s3 + (be3 - m3 * s3)

    w1a = jnp.concatenate([W1f, b1f[None, :], jnp.zeros((1, 128), f32)], axis=0)
    W2c = W2.astype(bf16)
    w3a = W3f[0:256].astype(bf16)
    w3b = W3f[256:512].astype(bf16)
    W4c = W4.astype(bf16)
    wf1c = Wf1[0:64]
    wf1s = Wf1[64:128]
    wf1l = Wf1[128:129]

    row2 = lambda a: a.reshape(1, -1)

    grid = N // NB
    full = lambda shape: pl.BlockSpec(shape, lambda i: (0, 0))
    out1, out2 = pl.pallas_call(
        _encoder_kernel,
        grid=(grid,),
        compiler_params=pltpu.CompilerParams(
            dimension_semantics=("parallel",)),
        in_specs=[
            pl.BlockSpec((NBP, 8), lambda i: (i, 0)),
            pl.BlockSpec((NBP, 1), lambda i: (i, 0)),
            pl.BlockSpec((NB, 1), lambda i: (i, 0)),
            pl.BlockSpec((NB, 1), lambda i: (i, 0)),
            pl.BlockSpec((NB, 128), lambda i: (i, 0)),
            pl.BlockSpec((NB, 128), lambda i: (i, 0)),
            full((8, 128)),
            full((128, 256)), full((1, 256)),
            full((256, 256)), full((256, 256)), full((1, 256)),
            full((256, 128)), full((1, 128)),
            pl.BlockSpec((NB, 64), lambda i: (i, 0)),
            pl.BlockSpec((NB, 64), lambda i: (i, 0)),
            full((64, 128)), full((64, 128)),
            full((1, 128)), full((1, 128)),
            full((1, 128)), full((1, 128)),
            full((128, 128)), full((1, 128)),
        ],
        out_specs=[
            pl.BlockSpec((NB, 128), lambda i: (i, 0)),
            pl.BlockSpec((NB, 128), lambda i: (i, 0)),
        ],
        out_shape=[
            jax.ShapeDtypeStruct((N, 128), f32),
            jax.ShapeDtypeStruct((N, 128), f32),
        ],
    )(pv, vm, sl, hasl, e1, e2,
      w1a, W2c, row2(b2), w3a, w3b, row2(b3f), W4c, row2(b4),
      ffc, ffs, wf1c, wf1s, wf1l, row2(bf1), row2(lng), row2(lnb), Wf2,
      row2(bf2))

    return out1.reshape(BS, M, 128), out2.reshape(BS, M, 128)


# speedup vs baseline: 1.0223x; 1.0223x over previous
import math

import jax
import jax.numpy as jnp
import numpy as np
from jax import lax
from jax.experimental import pallas as pl
from jax.experimental.pallas import tpu as pltpu
from jax.experimental.pallas import tpu_sc as plsc

BS, M, P = 32, 256, 20
PP = 24
N = BS * M
NB = 256
NBP = NB * PP

_SC_NC, _SC_NS = 2, 16
_SC_NW = _SC_NC * _SC_NS
_GB = N // _SC_NW
_GCH = 128

_TY_C = np.arange(24) // 8
_RT_C = (np.arange(24) // 4) % 2
_TL_C = np.arange(24) % 4
_TLI_C = np.where(_TY_C == 1, np.array([2, 0, 0, 3])[_TL_C], _TL_C)


def _embed_gather_kernel(table_hbm, gidx_hbm, out_hbm, idx_v, rows_v, sem):
    wid = lax.axis_index("s") * _SC_NC + lax.axis_index("c")
    base = wid * _GB
    for c in range(_GB // _GCH):
        off = base + c * _GCH
        pltpu.sync_copy(gidx_hbm.at[pl.ds(off, _GCH)], idx_v)
        pltpu.async_copy(table_hbm.at[idx_v], rows_v, sem).wait()
        pltpu.sync_copy(rows_v, out_hbm.at[pl.ds(off, _GCH)])


def _encoder_kernel(pv_ref, vm_ref, sl_ref, hasl_ref, e12_ref,
                    w1a_ref, w2_ref, b2_ref,
                    w3a_ref, w3b_ref, b3_ref, w4_ref, b4_ref,
                    ffc_ref, ffs_ref, wf1c_ref, wf1s_ref, wf1l_ref, bf1_ref,
                    lng_ref, lnb_ref, wf2_ref, bf2_ref,
                    out1_ref, out2_ref):
    f32 = jnp.float32
    bf16 = jnp.bfloat16

    h1 = jnp.dot(pv_ref[...], w1a_ref[...], preferred_element_type=f32)
    h1 = jax.nn.relu(h1)

    h = jnp.dot(h1.astype(bf16), w2_ref[...], preferred_element_type=f32)
    h = h + b2_ref[...]
    vm = vm_ref[...]
    hm = h * vm

    io8 = jax.lax.broadcasted_iota(jnp.int32, (NB, 8, 1), 1)
    pbs = jnp.where(io8 >= P - 16, f32(-1e30), f32(0.0))

    hm3 = hm.reshape(NB, PP, 256)
    m8 = jnp.maximum(jnp.maximum(hm3[:, 0:8], hm3[:, 8:16]),
                     hm3[:, 16:24] + pbs)
    pooled = jnp.max(m8, axis=1)

    q = jnp.dot(pooled.astype(bf16), w3b_ref[...], preferred_element_type=f32)
    q = q + b3_ref[...]
    qb = jnp.broadcast_to(q.reshape(NB, 1, 256), (NB, PP, 256)).reshape(NBP, 256)
    t = jnp.dot(hm.astype(bf16), w3a_ref[...], preferred_element_type=f32) + qb
    t = jax.nn.relu(t)

    z = jnp.dot(t.astype(bf16), w4_ref[...], preferred_element_type=f32)
    z = (z + b4_ref[...]) * vm
    z3 = z.reshape(NB, PP, 128)
    z8 = jnp.maximum(jnp.maximum(z3[:, 0:8], z3[:, 8:16]),
                     z3[:, 16:24] + pbs)
    x_poly = jnp.max(z8, axis=1)

    sl = sl_ref[...]
    fh = (jnp.dot(ffc_ref[...], wf1c_ref[...], preferred_element_type=f32)
          + jnp.dot(ffs_ref[...], wf1s_ref[...], preferred_element_type=f32)
          + sl * wf1l_ref[...] + bf1_ref[...])
    mu = jnp.mean(fh, axis=-1, keepdims=True)
    var = jnp.mean((fh - mu) ** 2, axis=-1, keepdims=True)
    fh = (fh - mu) / jnp.sqrt(var + 1e-5) * lng_ref[...] + lnb_ref[...]
    fh = jax.nn.relu(fh)
    fh = jnp.dot(fh, wf2_ref[...], preferred_element_type=f32) + bf2_ref[...]
    xs = fh * hasl_ref[...]

    base = x_poly + xs
    e12 = e12_ref[...]
    out1_ref[...] = base + e12[:, 0:128]
    out2_ref[...] = base + e12[:, 128:256]


def kernel(polygon_center, polygon_speed_limit, point_position, point_vector,
           point_orientation, polygon_type, polygon_on_route, polygon_tl_status,
           polygon_has_speed_limit, valid_mask, W1, b1, g1, be1, m1, v1, W2, b2,
           W3, b3, g3, be3, m3, v3, W4, b4, freqs, Wf1, bf1, lng, lnb, Wf2, bf2,
           type_table, route_table, tl_table):
    f32 = jnp.float32
    bf16 = jnp.bfloat16

    o = point_orientation[:, :, 0]
    ones = jnp.ones(o.shape + (1,), f32)
    pv = jnp.concatenate(
        [point_position[:, :, 0] - polygon_center[:, :, None, :2],
         point_vector[:, :, 0],
         jnp.cos(o)[..., None], jnp.sin(o)[..., None], ones],
        axis=-1).reshape(N, P, 7)
    pv = jnp.pad(pv, ((0, 0), (0, PP - P), (0, 1))).reshape(N * PP, 8)
    vm = jnp.pad(valid_mask.astype(f32).reshape(N, P),
                 ((0, 0), (0, PP - P))).reshape(N * PP, 1)
    sl = polygon_speed_limit.reshape(N, 1)
    xf = sl * (freqs * (2.0 * math.pi))[None, :]
    ffc = jnp.cos(xf)
    ffs = jnp.sin(xf)
    hasl = polygon_has_speed_limit.reshape(N, 1).astype(f32)

    ty = polygon_type.reshape(N).astype(jnp.int32)
    rt = polygon_on_route.reshape(N).astype(jnp.int32)
    tl = polygon_tl_status.reshape(N).astype(jnp.int32)
    gidx = ty * 8 + rt * 4 + tl
    etab = jnp.concatenate([
        type_table[_TY_C] + route_table[_RT_C] + tl_table[_TL_C],
        type_table[_TY_C] + route_table[_RT_C] + tl_table[_TLI_C],
    ], axis=1)
    e12 = pl.kernel(
        _embed_gather_kernel,
        out_type=jax.ShapeDtypeStruct((N, 256), f32),
        mesh=plsc.VectorSubcoreMesh(core_axis_name="c", subcore_axis_name="s"),
        scratch_types=[
            pltpu.VMEM((_GCH,), jnp.int32),
            pltpu.VMEM((_GCH, 256), f32),
            pltpu.SemaphoreType.DMA,
        ],
    )(etab, gidx)

    s1 = g1 / jnp.sqrt(v1 + 1e-5)
    W1f = W1 * s1
    b1f = b1 * s1 + (be1 - m1 * s1)
    s3 = g3 / jnp.sqrt(v3 + 1e-5)
    W3f = W3 * s3
    b3f = b3 * s3 + (be3 - m3 * s3)

    w1a = jnp.concatenate([W1f, b1f[None, :], jnp.zeros((1, 128), f32)], axis=0)
    W2c = W2.astype(bf16)
    w3a = W3f[0:256].astype(bf16)
    w3b = W3f[256:512].astype(bf16)
    W4c = W4.astype(bf16)
    wf1c = Wf1[0:64]
    wf1s = Wf1[64:128]
    wf1l = Wf1[128:129]

    row2 = lambda a: a.reshape(1, -1)

    grid = N // NB
    full = lambda shape: pl.BlockSpec(shape, lambda i: (0, 0))
    out1, out2 = pl.pallas_call(
        _encoder_kernel,
        grid=(grid,),
        compiler_params=pltpu.CompilerParams(
            dimension_semantics=("parallel",)),
        in_specs=[
            pl.BlockSpec((NBP, 8), lambda i: (i, 0)),
            pl.BlockSpec((NBP, 1), lambda i: (i, 0)),
            pl.BlockSpec((NB, 1), lambda i: (i, 0)),
            pl.BlockSpec((NB, 1), lambda i: (i, 0)),
            pl.BlockSpec((NB, 256), lambda i: (i, 0)),
            full((8, 128)),
            full((128, 256)), full((1, 256)),
            full((256, 256)), full((256, 256)), full((1, 256)),
            full((256, 128)), full((1, 128)),
            pl.BlockSpec((NB, 64), lambda i: (i, 0)),
            pl.BlockSpec((NB, 64), lambda i: (i, 0)),
            full((64, 128)), full((64, 128)),
            full((1, 128)), full((1, 128)),
            full((1, 128)), full((1, 128)),
            full((128, 128)), full((1, 128)),
        ],
        out_specs=[
            pl.BlockSpec((NB, 128), lambda i: (i, 0)),
            pl.BlockSpec((NB, 128), lambda i: (i, 0)),
        ],
        out_shape=[
            jax.ShapeDtypeStruct((N, 128), f32),
            jax.ShapeDtypeStruct((N, 128), f32),
        ],
    )(pv, vm, sl, hasl, e12,
      w1a, W2c, row2(b2), w3a, w3b, row2(b3f), W4c, row2(b4),
      ffc, ffs, wf1c, wf1s, wf1l, row2(bf1), row2(lng), row2(lnb), Wf2,
      row2(bf2))

    return out1.reshape(BS, M, 128), out2.reshape(BS, M, 128)


# speedup vs baseline: 1.1514x; 1.1263x over previous
import math

import jax
import jax.numpy as jnp
import numpy as np
from jax import lax
from jax.experimental import pallas as pl
from jax.experimental.pallas import tpu as pltpu
from jax.experimental.pallas import tpu_sc as plsc

BS, M, P = 32, 256, 20
PP = 24
N = BS * M
NB = 256
NBP = NB * PP

_SC_NC, _SC_NS = 2, 16
_SC_NW = _SC_NC * _SC_NS
_GB = N // _SC_NW
_GCH = 128

_TY_C = np.arange(24) // 8
_RT_C = (np.arange(24) // 4) % 2
_TL_C = np.arange(24) % 4
_TLI_C = np.where(_TY_C == 1, np.array([2, 0, 0, 3])[_TL_C], _TL_C)


def _embed_gather_kernel(table_hbm, gidx_hbm, out_hbm, idx_v, rows_v, sem):
    wid = lax.axis_index("s") * _SC_NC + lax.axis_index("c")
    base = wid * _GB
    for c in range(_GB // _GCH):
        off = base + c * _GCH
        pltpu.sync_copy(gidx_hbm.at[pl.ds(off, _GCH)], idx_v)
        pltpu.async_copy(table_hbm.at[idx_v], rows_v, sem).wait()
        pltpu.sync_copy(rows_v, out_hbm.at[pl.ds(off, _GCH)])


def _encoder_kernel(pv_ref, vm_ref, sl_ref, hasl_ref, e12_ref,
                    w1a_ref, w2_ref, b2_ref,
                    w3a_ref, w3b_ref, b3_ref, w4_ref, b4_ref,
                    ffc_ref, ffs_ref, wf1c_ref, wf1s_ref, wf1l_ref, bf1_ref,
                    lng_ref, lnb_ref, wf2_ref, bf2_ref,
                    out1_ref, out2_ref):
    f32 = jnp.float32
    bf16 = jnp.bfloat16

    h1 = jnp.dot(pv_ref[...], w1a_ref[...], preferred_element_type=f32)
    h1 = jax.nn.relu(h1)

    h = jnp.dot(h1.astype(bf16), w2_ref[...], preferred_element_type=f32)
    h = h + b2_ref[...]
    vm = vm_ref[...]
    hm = h * vm

    io8 = jax.lax.broadcasted_iota(jnp.int32, (NB, 8, 1), 1)
    pbs = jnp.where(io8 >= P - 16, f32(-1e30), f32(0.0))

    hm3 = hm.reshape(NB, PP, 256)
    m8 = jnp.maximum(jnp.maximum(hm3[:, 0:8], hm3[:, 8:16]),
                     hm3[:, 16:24] + pbs)
    pooled = jnp.max(m8, axis=1)

    q = jnp.dot(pooled.astype(bf16), w3b_ref[...], preferred_element_type=f32)
    q = q + b3_ref[...]
    qb = jnp.broadcast_to(q.reshape(NB, 1, 256), (NB, PP, 256)).reshape(NBP, 256)
    t = jnp.dot(hm.astype(bf16), w3a_ref[...], preferred_element_type=f32) + qb
    t = jax.nn.relu(t)

    z = jnp.dot(t.astype(bf16), w4_ref[...], preferred_element_type=f32)
    z = (z + b4_ref[...]) * vm
    z3 = z.reshape(NB, PP, 128)
    z8 = jnp.maximum(jnp.maximum(z3[:, 0:8], z3[:, 8:16]),
                     z3[:, 16:24] + pbs)
    x_poly = jnp.max(z8, axis=1)

    sl = sl_ref[...]
    fh = (jnp.dot(ffc_ref[...], wf1c_ref[...], preferred_element_type=f32)
          + jnp.dot(ffs_ref[...], wf1s_ref[...], preferred_element_type=f32)
          + sl * wf1l_ref[...] + bf1_ref[...])
    mu = jnp.mean(fh, axis=-1, keepdims=True)
    var = jnp.mean((fh - mu) ** 2, axis=-1, keepdims=True)
    fh = (fh - mu) / jnp.sqrt(var + 1e-5) * lng_ref[...] + lnb_ref[...]
    fh = jax.nn.relu(fh)
    fh = jnp.dot(fh, wf2_ref[...], preferred_element_type=f32) + bf2_ref[...]
    xs = fh * hasl_ref[...]

    base = x_poly + xs
    e12 = e12_ref[...]
    out1_ref[...] = base + e12[:, 0:128]
    out2_ref[...] = base + e12[:, 128:256]


def kernel(polygon_center, polygon_speed_limit, point_position, point_vector,
           point_orientation, polygon_type, polygon_on_route, polygon_tl_status,
           polygon_has_speed_limit, valid_mask, W1, b1, g1, be1, m1, v1, W2, b2,
           W3, b3, g3, be3, m3, v3, W4, b4, freqs, Wf1, bf1, lng, lnb, Wf2, bf2,
           type_table, route_table, tl_table):
    f32 = jnp.float32
    bf16 = jnp.bfloat16

    o = point_orientation[:, :, 0]
    ones = jnp.ones(o.shape + (1,), f32)
    pv = jnp.concatenate(
        [point_position[:, :, 0] - polygon_center[:, :, None, :2],
         point_vector[:, :, 0],
         jnp.cos(o)[..., None], jnp.sin(o)[..., None], ones],
        axis=-1).reshape(N, P, 7)
    pv = jnp.pad(pv, ((0, 0), (0, PP - P), (0, 1))).reshape(N * PP, 8)
    pv = pv.astype(bf16)
    vm = jnp.pad(valid_mask.astype(f32).reshape(N, P),
                 ((0, 0), (0, PP - P))).reshape(N * PP, 1)
    sl = polygon_speed_limit.reshape(N, 1)
    xf = sl * (freqs * (2.0 * math.pi))[None, :]
    ffc = jnp.cos(xf).astype(bf16)
    ffs = jnp.sin(xf).astype(bf16)
    hasl = polygon_has_speed_limit.reshape(N, 1).astype(f32)

    ty = polygon_type.reshape(N).astype(jnp.int32)
    rt = polygon_on_route.reshape(N).astype(jnp.int32)
    tl = polygon_tl_status.reshape(N).astype(jnp.int32)
    gidx = ty * 8 + rt * 4 + tl
    etab = jnp.concatenate([
        type_table[_TY_C] + route_table[_RT_C] + tl_table[_TL_C],
        type_table[_TY_C] + route_table[_RT_C] + tl_table[_TLI_C],
    ], axis=1)
    e12 = pl.kernel(
        _embed_gather_kernel,
        out_type=jax.ShapeDtypeStruct((N, 256), f32),
        mesh=plsc.VectorSubcoreMesh(core_axis_name="c", subcore_axis_name="s"),
        scratch_types=[
            pltpu.VMEM((_GCH,), jnp.int32),
            pltpu.VMEM((_GCH, 256), f32),
            pltpu.SemaphoreType.DMA,
        ],
    )(etab, gidx)

    s1 = g1 / jnp.sqrt(v1 + 1e-5)
    W1f = W1 * s1
    b1f = b1 * s1 + (be1 - m1 * s1)
    s3 = g3 / jnp.sqrt(v3 + 1e-5)
    W3f = W3 * s3
    b3f = b3 * s3 + (be3 - m3 * s3)

    w1a = jnp.concatenate([W1f, b1f[None, :],
                           jnp.zeros((1, 128), f32)], axis=0).astype(bf16)
    W2c = W2.astype(bf16)
    w3a = W3f[0:256].astype(bf16)
    w3b = W3f[256:512].astype(bf16)
    W4c = W4.astype(bf16)
    wf1c = Wf1[0:64].astype(bf16)
    wf1s = Wf1[64:128].astype(bf16)
    wf1l = Wf1[128:129]

    row2 = lambda a: a.reshape(1, -1)

    grid = N // NB
    full = lambda shape: pl.BlockSpec(shape, lambda i: (0, 0))
    out1, out2 = pl.pallas_call(
        _encoder_kernel,
        grid=(grid,),
        compiler_params=pltpu.CompilerParams(
            dimension_semantics=("parallel",)),
        in_specs=[
            pl.BlockSpec((NBP, 8), lambda i: (i, 0)),
            pl.BlockSpec((NBP, 1), lambda i: (i, 0)),
            pl.BlockSpec((NB, 1), lambda i: (i, 0)),
            pl.BlockSpec((NB, 1), lambda i: (i, 0)),
            pl.BlockSpec((NB, 256), lambda i: (i, 0)),
            full((8, 128)),
            full((128, 256)), full((1, 256)),
            full((256, 256)), full((256, 256)), full((1, 256)),
            full((256, 128)), full((1, 128)),
            pl.BlockSpec((NB, 64), lambda i: (i, 0)),
            pl.BlockSpec((NB, 64), lambda i: (i, 0)),
            full((64, 128)), full((64, 128)),
            full((1, 128)), full((1, 128)),
            full((1, 128)), full((1, 128)),
            full((128, 128)), full((1, 128)),
        ],
        out_specs=[
            pl.BlockSpec((NB, 128), lambda i: (i, 0)),
            pl.BlockSpec((NB, 128), lambda i: (i, 0)),
        ],
        out_shape=[
            jax.ShapeDtypeStruct((N, 128), f32),
            jax.ShapeDtypeStruct((N, 128), f32),
        ],
    )(pv, vm, sl, hasl, e12,
      w1a, W2c, row2(b2), w3a, w3b, row2(b3f), W4c, row2(b4),
      ffc, ffs, wf1c, wf1s, wf1l, row2(bf1), row2(lng), row2(lnb), Wf2,
      row2(bf2))

    return out1.reshape(BS, M, 128), out2.reshape(BS, M, 128)
